# Initial kernel scaffold; baseline (speedup 1.0000x reference)
import jax
import jax.numpy as jnp
from jax.experimental import pallas as pl


def kernel(x0, x1, edge_index0, edge_index1, edge_weight0, edge_weight1, W0_0, W1_0, W0_1, W1_1):
    raise NotImplementedError("write your pallas kernel here")


# speedup vs baseline: 5.9909x; 5.9909x over previous
import functools

import jax
import jax.numpy as jnp
from jax import lax
from jax.experimental import pallas as pl
from jax.experimental.pallas import tpu as pltpu
from jax.experimental.pallas import tpu_sc as plsc

N = 10000
E = 320000
D = 128
C = 16

NC = 2
NS = 16
CH = 128
NCH = 160
EPT = NCH * CH
EPAD = EPT * NS
NP = 10240
RPT = NP // NS


EB = 16


def _agg_body(width, xcat, srcs, dsts, ews, out,
              acc, esrc, edst, eew, rows, sem0, sem1):
    c = lax.axis_index("c")
    s = lax.axis_index("s")
    nfg = width // 16

    zvec = jnp.zeros((16,), jnp.float32)

    def zrow(r, carry):
        for j in range(nfg):
            rows[0, r, pl.ds(j * 16, 16)] = zvec
        return carry

    lax.fori_loop(0, CH, zrow, 0)
    row0 = s * RPT
    for k in range(RPT // CH):
        pltpu.sync_copy(rows.at[0], acc.at[pl.ds(row0 + k * CH, CH)])
    plsc.subcore_barrier()

    sems = [sem0, sem1]
    tile_base = (c * NS + s) * NCH

    def stage(st, carry):
        base = tile_base + st * EB
        pltpu.sync_copy(srcs.at[pl.ds(base, EB)], esrc)
        pltpu.sync_copy(dsts.at[pl.ds(base, EB)], edst)
        pltpu.sync_copy(ews.at[pl.ds(base, EB)], eew)
        for b in range(2):
            pltpu.async_copy(xcat.at[esrc.at[b]], rows.at[b], sems[b])

        def half(i, cc2):
            for b in range(2):
                ci = 2 * i + b
                pltpu.make_async_copy(xcat.at[esrc.at[ci]], rows.at[b],
                                      sems[b]).wait()

                def sgrp(g, cc):
                    wv = eew[ci, pl.ds(g * 16, 16)]
                    for k in range(16):
                        r = g * 16 + k
                        w = wv[k]
                        for j in range(nfg):
                            sl = pl.ds(j * 16, 16)
                            rows[b, r, sl] = rows[b, r, sl] * w
                    return cc

                lax.fori_loop(0, CH // 16, sgrp, 0)
                pltpu.sync_copy(rows.at[b], acc.at[edst.at[ci]], add=True)

                nci = ci + 2

                @pl.when(nci < EB)
                def _():
                    pltpu.async_copy(xcat.at[esrc.at[nci]], rows.at[b],
                                     sems[b])
            return cc2

        lax.fori_loop(0, EB // 2, half, 0)
        return carry

    lax.fori_loop(0, NCH // EB, stage, 0)

    plsc.subcore_barrier()
    pltpu.sync_copy(acc.at[pl.ds(s * RPT, RPT)],
                    out.at[pl.ds(c * NP + s * RPT, RPT)])


def _make_agg(width):
    mesh = plsc.VectorSubcoreMesh(core_axis_name="c", subcore_axis_name="s",
                                  num_cores=NC, num_subcores=NS)
    return pl.kernel(
        functools.partial(_agg_body, width),
        out_type=jax.ShapeDtypeStruct((2 * NP, width), jnp.float32),
        mesh=mesh,
        compiler_params=pltpu.CompilerParams(
            use_tc_tiling_on_sc=(width == D)),
        scratch_types=[
            pltpu.VMEM_SHARED((NP, width), jnp.float32),
            pltpu.VMEM((EB, CH), jnp.int32),
            pltpu.VMEM((EB, CH), jnp.int32),
            pltpu.VMEM((EB, CH), jnp.float32),
            pltpu.VMEM((2, CH, width), jnp.float32),
            pltpu.SemaphoreType.DMA,
            pltpu.SemaphoreType.DMA,
        ],
        name=f"mgcn_agg{width}",
    )


_agg128 = _make_agg(D)
_agg16 = _make_agg(C)


def _tc_body(x_ref, w0_ref, w1_ref, o_ref):
    t = x_ref[...]
    h = jnp.maximum(jnp.dot(t, w0_ref[0], preferred_element_type=jnp.float32),
                    0.0)
    o_ref[...] = jnp.dot(h, w1_ref[0],
                         preferred_element_type=jnp.float32) * 0.5


_BLK = 1024


def _tc_transform(xagg, w0s, w1s):
    grid = (2 * NP // _BLK,)
    per = NP // _BLK
    return pl.pallas_call(
        _tc_body,
        grid=grid,
        in_specs=[
            pl.BlockSpec((_BLK, D), lambda p: (p, 0)),
            pl.BlockSpec((1, D, D), lambda p: (p // per, 0, 0)),
            pl.BlockSpec((1, D, C), lambda p: (p // per, 0, 0)),
        ],
        out_specs=pl.BlockSpec((_BLK, C), lambda p: (p, 0)),
        out_shape=jax.ShapeDtypeStruct((2 * NP, C), jnp.float32),
    )(xagg, w0s, w1s)


def _prep_edges(edge_index, edge_weight, branch):
    src = jnp.pad(edge_index[0], (0, EPAD - E)) + branch * NP
    dst = jnp.pad(edge_index[1], (0, EPAD - E))
    ew = jnp.pad(edge_weight, (0, EPAD - E))
    return (src.reshape(NS, NCH, CH), dst.reshape(NS, NCH, CH),
            ew.reshape(NS, NCH, CH))


def kernel(x0, x1, edge_index0, edge_index1, edge_weight0, edge_weight1,
           W0_0, W1_0, W0_1, W1_1):
    s0, d0, w0 = _prep_edges(edge_index0, edge_weight0, 0)
    s1, d1, w1 = _prep_edges(edge_index1, edge_weight1, 1)
    srcs = jnp.concatenate([s0, s1]).reshape(2 * NS * NCH, CH)
    dsts = jnp.concatenate([d0, d1]).reshape(2 * NS * NCH, CH)
    ews = jnp.concatenate([w0, w1]).reshape(2 * NS * NCH, CH)

    xcat = jnp.zeros((2 * NP, D), jnp.float32)
    xcat = xcat.at[:N].set(x0).at[NP:NP + N].set(x1)
    xagg = _agg128(xcat, srcs, dsts, ews)
    y = _tc_transform(xagg, jnp.stack([W0_0, W0_1]),
                      jnp.stack([W1_0, W1_1]))
    parts = _agg16(y, srcs, dsts, ews)
    return parts[:N] + parts[NP:NP + N]
